# Initial kernel scaffold; baseline (speedup 1.0000x reference)
import jax
import jax.numpy as jnp
from jax.experimental import pallas as pl


def kernel(inputs, targets, unarys, topk, num_unary):
    raise NotImplementedError("write your pallas kernel here")


# speedup vs baseline: 33.7293x; 33.7293x over previous
import jax
import jax.numpy as jnp
from jax import lax
from jax.experimental import pallas as pl
from jax.experimental.pallas import tpu as pltpu

_B, _C, _H, _W = 8, 3, 512, 512
_HW = _H * _W
_ONE_BITS = 0x3F800000


def _tc_body(topk_ref, num_unary_ref, x_ref, tgt_ref, u_ref, out_ref):
    b = pl.program_id(0)

    @pl.when(b == 0)
    def _init():
        out_ref[0, 0] = 0.0
        out_ref[0, 1] = 0.0
        out_ref[0, 2] = 0.0

    bits = lax.bitcast_convert_type(u_ref[0], jnp.int32)
    k = topk_ref[b]

    def _vstep(_, st):
        lo, hi, cnt_hi = st
        mid = lo + (hi - lo) // 2
        cnt = jnp.sum((bits > mid).astype(jnp.int32))
        pred = cnt < k
        return (jnp.where(pred, lo, mid),
                jnp.where(pred, mid, hi),
                jnp.where(pred, cnt, cnt_hi))

    _, t, cnt_gt = lax.fori_loop(
        0, 30, _vstep, (jnp.int32(-1), jnp.int32(_ONE_BITS), jnp.int32(0)))

    extra = k - cnt_gt
    tie = bits == t
    idxmat = (lax.broadcasted_iota(jnp.int32, (_H, _W), 0) * _W
              + lax.broadcasted_iota(jnp.int32, (_H, _W), 1))

    def _istep(_, st):
        lo, hi = st
        mid = lo + (hi - lo) // 2
        cnt = jnp.sum((tie & (idxmat <= mid)).astype(jnp.int32))
        pred = cnt >= extra
        return jnp.where(pred, lo, mid), jnp.where(pred, mid, hi)

    _, m = lax.fori_loop(0, 18, _istep,
                         (jnp.int32(-1), jnp.int32(_HW - 1)))

    x0 = x_ref[0, 0]
    x1 = x_ref[0, 1]
    x2 = x_ref[0, 2]
    tgt = tgt_ref[0]
    mx = jnp.maximum(x0, jnp.maximum(x1, x2))
    se = jnp.exp(x0 - mx) + jnp.exp(x1 - mx) + jnp.exp(x2 - mx)
    lse = jnp.log(se) + mx
    xt = jnp.where(tgt == 0, x0, x1)
    nll = jnp.where(tgt < 2, lse - xt, 0.0)

    fg_mask = (bits > t) | (tie & (idxmat <= m))
    out_ref[0, 0] += jnp.sum(nll)
    out_ref[0, 1] += jnp.sum(jnp.where(fg_mask, nll, 0.0))

    @pl.when(b == _B - 1)
    def _fin():
        s_nu = lax.fori_loop(
            0, _B, lambda i, a: a + num_unary_ref[i], jnp.int32(0))
        s_tk = lax.fori_loop(
            0, _B, lambda i, a: a + topk_ref[i], jnp.int32(0))
        denom_bg = (jnp.int32(_B * _HW) - s_nu + 1).astype(jnp.float32)
        out_ref[0, 2] = 0.5 * (out_ref[0, 0] / denom_bg
                               + out_ref[0, 1] / s_tk.astype(jnp.float32))


def kernel(inputs, targets, unarys, topk, num_unary):
    out = pl.pallas_call(
        _tc_body,
        grid=(_B,),
        in_specs=[
            pl.BlockSpec(memory_space=pltpu.SMEM),
            pl.BlockSpec(memory_space=pltpu.SMEM),
            pl.BlockSpec((1, _C, _H, _W), lambda b: (b, 0, 0, 0)),
            pl.BlockSpec((1, _H, _W), lambda b: (b, 0, 0)),
            pl.BlockSpec((1, _H, _W), lambda b: (b, 0, 0)),
        ],
        out_specs=pl.BlockSpec(memory_space=pltpu.SMEM),
        out_shape=jax.ShapeDtypeStruct((1, 4), jnp.float32),
    )(topk, num_unary, inputs, targets, unarys)
    return out[0, 2]
